# baseline (speedup 1.0000x reference)
import functools

import jax
import jax.numpy as jnp
from jax import lax
from jax.experimental import pallas as pl
from jax.experimental.pallas import tpu as pltpu
from jax.experimental.pallas import tpu_sc as plsc

N = 10000
E = 160000
RAW = 119
H = 300
C = 2
B = 10

HALF = 150
HP = 160
NC = 2
NS = 16
EPT = E // NS
K = 125
NCH = EPT // K
NG = 2
GCH = NCH // NG
RPT = N // NS
BN = 1000
NB = N // BN

_f32 = jnp.float32
_bf16 = jnp.bfloat16


def _sc_body(src3, dst3, zrows, m0, m1, agg0, agg1,
             src_v, dst_v, rows0, rows1, rows2, rows3, rows4, shared,
             sem0, sem1, sem2, sem3, sem4):
  c = lax.axis_index("c")
  s = lax.axis_index("s")

  pltpu.sync_copy(zrows, shared.at[pl.ds(s * RPT, RPT)])
  plsc.subcore_barrier()

  RING = 5

  def accumulate(m_ref):
    bufs = (rows0, rows1, rows2, rows3, rows4)
    sems = (sem0, sem1, sem2, sem3, sem4)
    for g in range(NG):
      pltpu.sync_copy(src3.at[s, pl.ds(g * GCH, GCH)], src_v)
      pltpu.sync_copy(dst3.at[s, pl.ds(g * GCH, GCH)], dst_v)

      for u in range(RING - 1):
        pltpu.async_copy(m_ref.at[src_v.at[u]], bufs[u], sems[u])

      def body(jj, carry):
        for u in range(RING):
          j = jj * RING + u
          b, e = bufs[u], sems[u]
          pltpu.make_async_copy(m_ref.at[src_v.at[j]], b, e).wait()

          @pl.when(j + RING - 1 < GCH)
          def _():
            nb = bufs[(u + RING - 1) % RING]
            ne = sems[(u + RING - 1) % RING]
            pltpu.async_copy(m_ref.at[src_v.at[j + RING - 1]], nb, ne)

          pltpu.sync_copy(b, shared.at[dst_v.at[j]], add=True)
        return carry

      lax.fori_loop(0, GCH // RING, body, 0)

  @pl.when(c == 0)
  def _():
    accumulate(m0)

  @pl.when(c == 1)
  def _():
    accumulate(m1)

  plsc.subcore_barrier()

  def writeback(agg_ref):
    pltpu.sync_copy(shared.at[pl.ds(s * RPT, RPT)],
                    agg_ref.at[pl.ds(s * RPT, RPT)])

  @pl.when(c == 0)
  def _():
    writeback(agg0)

  @pl.when(c == 1)
  def _():
    writeback(agg1)


@functools.cache
def _make_scatter_gather():
  return pl.kernel(
      _sc_body,
      out_type=(
          jax.ShapeDtypeStruct((N, HP), _bf16),
          jax.ShapeDtypeStruct((N, HP), _bf16),
      ),
      mesh=plsc.VectorSubcoreMesh(core_axis_name="c", subcore_axis_name="s",
                                  num_cores=NC, num_subcores=NS),
      compiler_params=pltpu.CompilerParams(use_tc_tiling_on_sc=False),
      scratch_types=[
          pltpu.VMEM((GCH, K), jnp.int32),
          pltpu.VMEM((GCH, K), jnp.int32),
          pltpu.VMEM((K, HP), _bf16),
          pltpu.VMEM((K, HP), _bf16),
          pltpu.VMEM((K, HP), _bf16),
          pltpu.VMEM((K, HP), _bf16),
          pltpu.VMEM((K, HP), _bf16),
          pltpu.VMEM_SHARED((N, HP), _bf16),
          pltpu.SemaphoreType.DMA,
          pltpu.SemaphoreType.DMA,
          pltpu.SemaphoreType.DMA,
          pltpu.SemaphoreType.DMA,
          pltpu.SemaphoreType.DMA,
      ],
  )


def _scatter_gather(src3, dst3, zrows, m0, m1):
  return _make_scatter_gather()(src3, dst3, zrows, m0, m1)


def _lift_msg_body(x, wl, bl, w0, b0, w1, b1, m0, m1):
  h = jnp.dot(x[...], wl[...], preferred_element_type=_f32) + bl[...]
  m0[...] = jax.nn.relu(jnp.dot(h, w0[...], preferred_element_type=_f32)
                        + b0[...]).astype(_bf16)
  m1[...] = jax.nn.relu(jnp.dot(h, w1[...], preferred_element_type=_f32)
                        + b1[...]).astype(_bf16)


def _mid_body(a0, a1, wa, wb, bo, w0, b0, w1, b1, m0, m1):
  af0 = a0[...].astype(_f32)
  af1 = a1[...].astype(_f32)
  h = jax.nn.relu(jnp.dot(af0, wa[...], preferred_element_type=_f32)
                  + jnp.dot(af1, wb[...], preferred_element_type=_f32)
                  + bo[...])
  m0[...] = jax.nn.relu(jnp.dot(h, w0[...], preferred_element_type=_f32)
                        + b0[...]).astype(_bf16)
  m1[...] = jax.nn.relu(jnp.dot(h, w1[...], preferred_element_type=_f32)
                        + b1[...]).astype(_bf16)


def _readout_body(a0, a1, wa, wb, bo, wro, bro, gid, out):
  af0 = a0[...].astype(_f32)
  af1 = a1[...].astype(_f32)
  h = jax.nn.relu(jnp.dot(af0, wa[...], preferred_element_type=_f32)
                  + jnp.dot(af1, wb[...], preferred_element_type=_f32)
                  + bo[...])
  nl = jnp.dot(h, wro[...], preferred_element_type=_f32) + bro[...]
  gids = lax.broadcasted_iota(jnp.int32, (BN, 16), 1)
  onehot = (gid[...] == gids).astype(_f32)
  contrib = lax.dot_general(onehot, nl, (((0,), (0,)), ((), ())),
                            preferred_element_type=_f32)

  @pl.when(pl.program_id(0) == 0)
  def _():
    out[...] = jnp.zeros_like(out)

  out[...] += contrib


def _full(shape):
  return pl.BlockSpec(shape, lambda i: (0,) * len(shape))


def _rows(width):
  return pl.BlockSpec((BN, width), lambda i: (i, 0))


_lift_msg = pl.pallas_call(
    _lift_msg_body,
    grid=(NB,),
    in_specs=[_rows(RAW), _full((RAW, H)), _full((1, H)),
              _full((H, HP)), _full((1, HP)), _full((H, HP)), _full((1, HP))],
    out_specs=[_rows(HP), _rows(HP)],
    out_shape=[jax.ShapeDtypeStruct((N, HP), _bf16)] * 2,
)

_mid = pl.pallas_call(
    _mid_body,
    grid=(NB,),
    in_specs=[_rows(HP), _rows(HP),
              _full((HP, H)), _full((HP, H)), _full((1, H)),
              _full((H, HP)), _full((1, HP)), _full((H, HP)), _full((1, HP))],
    out_specs=[_rows(HP), _rows(HP)],
    out_shape=[jax.ShapeDtypeStruct((N, HP), _bf16)] * 2,
)

_readout = pl.pallas_call(
    _readout_body,
    grid=(NB,),
    in_specs=[_rows(HP), _rows(HP),
              _full((HP, H)), _full((HP, H)), _full((1, H)),
              _full((H, 128)), _full((1, 128)), _rows(1)],
    out_specs=pl.BlockSpec((16, 128), lambda i: (0, 0)),
    out_shape=jax.ShapeDtypeStruct((16, 128), _f32),
)


def _split_cols(w, b):
  pad = jnp.zeros((H, HP - HALF), _f32)
  bpad = jnp.zeros((1, HP - HALF), _f32)
  w0 = jnp.concatenate([w[:, :HALF], pad], axis=1)
  w1 = jnp.concatenate([w[:, HALF:], pad], axis=1)
  b0 = jnp.concatenate([b[:HALF][None], bpad], axis=1)
  b1 = jnp.concatenate([b[HALF:][None], bpad], axis=1)
  return w0, b0, w1, b1


def _split_rows(w):
  pad = jnp.zeros((HP - HALF, H), _f32)
  wa = jnp.concatenate([w[:HALF, :], pad], axis=0)
  wb = jnp.concatenate([w[HALF:, :], pad], axis=0)
  return wa, wb


def kernel(x, edge_index, graph_ids, W_lift, b_lift, W_ro, b_ro,
           W_msg0, b_msg0, W_out0, b_out0,
           W_msg1, b_msg1, W_out1, b_out1,
           W_msg2, b_msg2, W_out2, b_out2):
  src3 = edge_index[0].reshape(NS, NCH, K)
  dst3 = edge_index[1].reshape(NS, NCH, K)
  zrows = jnp.zeros((RPT, HP), _bf16)

  w00, b00, w01, b01 = _split_cols(W_msg0, b_msg0)
  w10, b10, w11, b11 = _split_cols(W_msg1, b_msg1)
  w20, b20, w21, b21 = _split_cols(W_msg2, b_msg2)
  wa0, wb0 = _split_rows(W_out0)
  wa1, wb1 = _split_rows(W_out1)
  wa2, wb2 = _split_rows(W_out2)

  wro = jnp.zeros((H, 128), _f32).at[:, :C].set(W_ro)
  bro = jnp.zeros((1, 128), _f32).at[0, :C].set(b_ro)

  m0, m1 = _lift_msg(x, W_lift, b_lift.reshape(1, H),
                     w00, b00, w01, b01)
  a0, a1 = _scatter_gather(src3, dst3, zrows, m0, m1)

  m0, m1 = _mid(a0, a1, wa0, wb0, b_out0.reshape(1, H), w10, b10, w11, b11)
  a0, a1 = _scatter_gather(src3, dst3, zrows, m0, m1)

  m0, m1 = _mid(a0, a1, wa1, wb1, b_out1.reshape(1, H), w20, b20, w21, b21)
  a0, a1 = _scatter_gather(src3, dst3, zrows, m0, m1)

  acc = _readout(a0, a1, wa2, wb2, b_out2.reshape(1, H), wro, bro,
                 graph_ids.reshape(N, 1))
  return acc[:B, :C]


# speedup vs baseline: 1.0218x; 1.0218x over previous
import functools

import jax
import jax.numpy as jnp
from jax import lax
from jax.experimental import pallas as pl
from jax.experimental.pallas import tpu as pltpu
from jax.experimental.pallas import tpu_sc as plsc

N = 10000
E = 160000
RAW = 119
H = 300
C = 2
B = 10

HALF = 150
HP = 160
NC = 2
NS = 16
EPT = E // NS
K = 125
NCH = EPT // K
NG = 1
GCH = NCH // NG
RPT = N // NS
BN = 1000
NB = N // BN

_f32 = jnp.float32
_bf16 = jnp.bfloat16


def _sc_body(src3, dst3, zrows, m0, m1, agg0, agg1,
             src_v, dst_v, rows0, rows1, rows2, rows3, shared,
             sem0, sem1, sem2, sem3):
  c = lax.axis_index("c")
  s = lax.axis_index("s")

  pltpu.sync_copy(zrows, shared.at[pl.ds(s * RPT, RPT)])
  plsc.subcore_barrier()

  RING = 4

  def accumulate(m_ref):
    bufs = (rows0, rows1, rows2, rows3)
    sems = (sem0, sem1, sem2, sem3)
    for g in range(NG):
      pltpu.sync_copy(src3.at[s, pl.ds(g * GCH, GCH)], src_v)
      pltpu.sync_copy(dst3.at[s, pl.ds(g * GCH, GCH)], dst_v)

      for u in range(RING - 1):
        pltpu.async_copy(m_ref.at[src_v.at[u]], bufs[u], sems[u])

      def body(jj, carry):
        for u in range(RING):
          j = jj * RING + u
          b, e = bufs[u], sems[u]
          pltpu.make_async_copy(m_ref.at[src_v.at[j]], b, e).wait()

          @pl.when(j + RING - 1 < GCH)
          def _():
            nb = bufs[(u + RING - 1) % RING]
            ne = sems[(u + RING - 1) % RING]
            pltpu.async_copy(m_ref.at[src_v.at[j + RING - 1]], nb, ne)

          pltpu.sync_copy(b, shared.at[dst_v.at[j]], add=True)
        return carry

      lax.fori_loop(0, GCH // RING, body, 0)

  @pl.when(c == 0)
  def _():
    accumulate(m0)

  @pl.when(c == 1)
  def _():
    accumulate(m1)

  plsc.subcore_barrier()

  def writeback(agg_ref):
    pltpu.sync_copy(shared.at[pl.ds(s * RPT, RPT)],
                    agg_ref.at[pl.ds(s * RPT, RPT)])

  @pl.when(c == 0)
  def _():
    writeback(agg0)

  @pl.when(c == 1)
  def _():
    writeback(agg1)


@functools.cache
def _make_scatter_gather():
  return pl.kernel(
      _sc_body,
      out_type=(
          jax.ShapeDtypeStruct((N, HP), _bf16),
          jax.ShapeDtypeStruct((N, HP), _bf16),
      ),
      mesh=plsc.VectorSubcoreMesh(core_axis_name="c", subcore_axis_name="s",
                                  num_cores=NC, num_subcores=NS),
      compiler_params=pltpu.CompilerParams(use_tc_tiling_on_sc=False),
      scratch_types=[
          pltpu.VMEM((GCH, K), jnp.int32),
          pltpu.VMEM((GCH, K), jnp.int32),
          pltpu.VMEM((K, HP), _bf16),
          pltpu.VMEM((K, HP), _bf16),
          pltpu.VMEM((K, HP), _bf16),
          pltpu.VMEM((K, HP), _bf16),
          pltpu.VMEM_SHARED((N, HP), _bf16),
          pltpu.SemaphoreType.DMA,
          pltpu.SemaphoreType.DMA,
          pltpu.SemaphoreType.DMA,
          pltpu.SemaphoreType.DMA,
      ],
  )


def _scatter_gather(src3, dst3, zrows, m0, m1):
  return _make_scatter_gather()(src3, dst3, zrows, m0, m1)


def _lift_msg_body(x, wl, bl, w0, b0, w1, b1, m0, m1):
  h = jnp.dot(x[...], wl[...], preferred_element_type=_f32) + bl[...]
  m0[...] = jax.nn.relu(jnp.dot(h, w0[...], preferred_element_type=_f32)
                        + b0[...]).astype(_bf16)
  m1[...] = jax.nn.relu(jnp.dot(h, w1[...], preferred_element_type=_f32)
                        + b1[...]).astype(_bf16)


def _mid_body(a0, a1, wa, wb, bo, w0, b0, w1, b1, m0, m1):
  af0 = a0[...].astype(_f32)
  af1 = a1[...].astype(_f32)
  h = jax.nn.relu(jnp.dot(af0, wa[...], preferred_element_type=_f32)
                  + jnp.dot(af1, wb[...], preferred_element_type=_f32)
                  + bo[...])
  m0[...] = jax.nn.relu(jnp.dot(h, w0[...], preferred_element_type=_f32)
                        + b0[...]).astype(_bf16)
  m1[...] = jax.nn.relu(jnp.dot(h, w1[...], preferred_element_type=_f32)
                        + b1[...]).astype(_bf16)


def _readout_body(a0, a1, wa, wb, bo, wro, bro, gid, out):
  af0 = a0[...].astype(_f32)
  af1 = a1[...].astype(_f32)
  h = jax.nn.relu(jnp.dot(af0, wa[...], preferred_element_type=_f32)
                  + jnp.dot(af1, wb[...], preferred_element_type=_f32)
                  + bo[...])
  nl = jnp.dot(h, wro[...], preferred_element_type=_f32) + bro[...]
  gids = lax.broadcasted_iota(jnp.int32, (BN, 16), 1)
  onehot = (gid[...] == gids).astype(_f32)
  contrib = lax.dot_general(onehot, nl, (((0,), (0,)), ((), ())),
                            preferred_element_type=_f32)

  @pl.when(pl.program_id(0) == 0)
  def _():
    out[...] = jnp.zeros_like(out)

  out[...] += contrib


def _full(shape):
  return pl.BlockSpec(shape, lambda i: (0,) * len(shape))


def _rows(width):
  return pl.BlockSpec((BN, width), lambda i: (i, 0))


_lift_msg = pl.pallas_call(
    _lift_msg_body,
    grid=(NB,),
    in_specs=[_rows(RAW), _full((RAW, H)), _full((1, H)),
              _full((H, HP)), _full((1, HP)), _full((H, HP)), _full((1, HP))],
    out_specs=[_rows(HP), _rows(HP)],
    out_shape=[jax.ShapeDtypeStruct((N, HP), _bf16)] * 2,
)

_mid = pl.pallas_call(
    _mid_body,
    grid=(NB,),
    in_specs=[_rows(HP), _rows(HP),
              _full((HP, H)), _full((HP, H)), _full((1, H)),
              _full((H, HP)), _full((1, HP)), _full((H, HP)), _full((1, HP))],
    out_specs=[_rows(HP), _rows(HP)],
    out_shape=[jax.ShapeDtypeStruct((N, HP), _bf16)] * 2,
)

_readout = pl.pallas_call(
    _readout_body,
    grid=(NB,),
    in_specs=[_rows(HP), _rows(HP),
              _full((HP, H)), _full((HP, H)), _full((1, H)),
              _full((H, 128)), _full((1, 128)), _rows(1)],
    out_specs=pl.BlockSpec((16, 128), lambda i: (0, 0)),
    out_shape=jax.ShapeDtypeStruct((16, 128), _f32),
)


def _split_cols(w, b):
  pad = jnp.zeros((H, HP - HALF), _f32)
  bpad = jnp.zeros((1, HP - HALF), _f32)
  w0 = jnp.concatenate([w[:, :HALF], pad], axis=1)
  w1 = jnp.concatenate([w[:, HALF:], pad], axis=1)
  b0 = jnp.concatenate([b[:HALF][None], bpad], axis=1)
  b1 = jnp.concatenate([b[HALF:][None], bpad], axis=1)
  return w0, b0, w1, b1


def _split_rows(w):
  pad = jnp.zeros((HP - HALF, H), _f32)
  wa = jnp.concatenate([w[:HALF, :], pad], axis=0)
  wb = jnp.concatenate([w[HALF:, :], pad], axis=0)
  return wa, wb


def kernel(x, edge_index, graph_ids, W_lift, b_lift, W_ro, b_ro,
           W_msg0, b_msg0, W_out0, b_out0,
           W_msg1, b_msg1, W_out1, b_out1,
           W_msg2, b_msg2, W_out2, b_out2):
  src3 = edge_index[0].reshape(NS, NCH, K)
  dst3 = edge_index[1].reshape(NS, NCH, K)
  zrows = jnp.zeros((RPT, HP), _bf16)

  w00, b00, w01, b01 = _split_cols(W_msg0, b_msg0)
  w10, b10, w11, b11 = _split_cols(W_msg1, b_msg1)
  w20, b20, w21, b21 = _split_cols(W_msg2, b_msg2)
  wa0, wb0 = _split_rows(W_out0)
  wa1, wb1 = _split_rows(W_out1)
  wa2, wb2 = _split_rows(W_out2)

  wro = jnp.zeros((H, 128), _f32).at[:, :C].set(W_ro)
  bro = jnp.zeros((1, 128), _f32).at[0, :C].set(b_ro)

  m0, m1 = _lift_msg(x, W_lift, b_lift.reshape(1, H),
                     w00, b00, w01, b01)
  a0, a1 = _scatter_gather(src3, dst3, zrows, m0, m1)

  m0, m1 = _mid(a0, a1, wa0, wb0, b_out0.reshape(1, H), w10, b10, w11, b11)
  a0, a1 = _scatter_gather(src3, dst3, zrows, m0, m1)

  m0, m1 = _mid(a0, a1, wa1, wb1, b_out1.reshape(1, H), w20, b20, w21, b21)
  a0, a1 = _scatter_gather(src3, dst3, zrows, m0, m1)

  acc = _readout(a0, a1, wa2, wb2, b_out2.reshape(1, H), wro, bro,
                 graph_ids.reshape(N, 1))
  return acc[:B, :C]


# speedup vs baseline: 1.0223x; 1.0005x over previous
import functools

import jax
import jax.numpy as jnp
from jax import lax
from jax.experimental import pallas as pl
from jax.experimental.pallas import tpu as pltpu
from jax.experimental.pallas import tpu_sc as plsc

N = 10000
E = 160000
RAW = 119
H = 300
C = 2
B = 10

HALF = 150
HP = 160
NC = 2
NS = 16
EPT = E // NS
K = 125
NCH = EPT // K
NG = 1
GCH = NCH // NG
RPT = N // NS
BN = 1000
NB = N // BN

_f32 = jnp.float32
_bf16 = jnp.bfloat16


def _sc_body(src3, dst3, zrows, m0, m1, agg0, agg1,
             src_v, dst_v, rows0, rows1, rows2, rows3, shared,
             sem0, sem1, sem2, sem3):
  c = lax.axis_index("c")
  s = lax.axis_index("s")

  pltpu.sync_copy(zrows, shared.at[pl.ds(s * RPT, RPT)])
  plsc.subcore_barrier()

  RING = 4

  def accumulate(m_ref):
    bufs = (rows0, rows1, rows2, rows3)
    sems = (sem0, sem1, sem2, sem3)
    for g in range(NG):
      pltpu.sync_copy(src3.at[s, pl.ds(g * GCH, GCH)], src_v)
      pltpu.sync_copy(dst3.at[s, pl.ds(g * GCH, GCH)], dst_v)

      for u in range(RING - 1):
        pltpu.async_copy(m_ref.at[src_v.at[u]], bufs[u], sems[u])

      def body(jj, carry):
        for u in range(RING):
          j = jj * RING + u
          b, e = bufs[u], sems[u]
          pltpu.make_async_copy(m_ref.at[src_v.at[j]], b, e).wait()

          @pl.when(j + RING - 1 < GCH)
          def _():
            nb = bufs[(u + RING - 1) % RING]
            ne = sems[(u + RING - 1) % RING]
            pltpu.async_copy(m_ref.at[src_v.at[j + RING - 1]], nb, ne)

          pltpu.sync_copy(b, shared.at[dst_v.at[j]], add=True)
        return carry

      lax.fori_loop(0, GCH // RING, body, 0)

  @pl.when(c == 0)
  def _():
    accumulate(m0)

  @pl.when(c == 1)
  def _():
    accumulate(m1)

  plsc.subcore_barrier()

  def writeback(agg_ref):
    pltpu.sync_copy(shared.at[pl.ds(s * RPT, RPT)],
                    agg_ref.at[pl.ds(s * RPT, RPT)])

  @pl.when(c == 0)
  def _():
    writeback(agg0)

  @pl.when(c == 1)
  def _():
    writeback(agg1)


@functools.cache
def _make_scatter_gather():
  return pl.kernel(
      _sc_body,
      out_type=(
          jax.ShapeDtypeStruct((N, HP), _bf16),
          jax.ShapeDtypeStruct((N, HP), _bf16),
      ),
      mesh=plsc.VectorSubcoreMesh(core_axis_name="c", subcore_axis_name="s",
                                  num_cores=NC, num_subcores=NS),
      compiler_params=pltpu.CompilerParams(use_tc_tiling_on_sc=False),
      scratch_types=[
          pltpu.VMEM((GCH, K), jnp.int32),
          pltpu.VMEM((GCH, K), jnp.int32),
          pltpu.VMEM((K, HP), _bf16),
          pltpu.VMEM((K, HP), _bf16),
          pltpu.VMEM((K, HP), _bf16),
          pltpu.VMEM((K, HP), _bf16),
          pltpu.VMEM_SHARED((N, HP), _bf16),
          pltpu.SemaphoreType.DMA,
          pltpu.SemaphoreType.DMA,
          pltpu.SemaphoreType.DMA,
          pltpu.SemaphoreType.DMA,
      ],
  )


def _scatter_gather(src3, dst3, zrows, m0, m1):
  return _make_scatter_gather()(src3, dst3, zrows, m0, m1)


def _lift_msg_body(x, wl, bl, w0, b0, w1, b1, m0, m1):
  h = jnp.dot(x[...].astype(_bf16), wl[...],
              preferred_element_type=_f32) + bl[...]
  hb = h.astype(_bf16)
  m0[...] = jax.nn.relu(jnp.dot(hb, w0[...], preferred_element_type=_f32)
                        + b0[...]).astype(_bf16)
  m1[...] = jax.nn.relu(jnp.dot(hb, w1[...], preferred_element_type=_f32)
                        + b1[...]).astype(_bf16)


def _mid_body(a0, a1, wa, wb, bo, w0, b0, w1, b1, m0, m1):
  h = jax.nn.relu(jnp.dot(a0[...], wa[...], preferred_element_type=_f32)
                  + jnp.dot(a1[...], wb[...], preferred_element_type=_f32)
                  + bo[...])
  hb = h.astype(_bf16)
  m0[...] = jax.nn.relu(jnp.dot(hb, w0[...], preferred_element_type=_f32)
                        + b0[...]).astype(_bf16)
  m1[...] = jax.nn.relu(jnp.dot(hb, w1[...], preferred_element_type=_f32)
                        + b1[...]).astype(_bf16)


def _readout_body(a0, a1, wa, wb, bo, wro, bro, gid, out):
  h = jax.nn.relu(jnp.dot(a0[...], wa[...], preferred_element_type=_f32)
                  + jnp.dot(a1[...], wb[...], preferred_element_type=_f32)
                  + bo[...])
  nl = jnp.dot(h, wro[...], preferred_element_type=_f32) + bro[...]
  gids = lax.broadcasted_iota(jnp.int32, (BN, 16), 1)
  onehot = (gid[...] == gids).astype(_f32)
  contrib = lax.dot_general(onehot, nl, (((0,), (0,)), ((), ())),
                            preferred_element_type=_f32)

  @pl.when(pl.program_id(0) == 0)
  def _():
    out[...] = jnp.zeros_like(out)

  out[...] += contrib


def _full(shape):
  return pl.BlockSpec(shape, lambda i: (0,) * len(shape))


def _rows(width):
  return pl.BlockSpec((BN, width), lambda i: (i, 0))


_lift_msg = pl.pallas_call(
    _lift_msg_body,
    grid=(NB,),
    in_specs=[_rows(RAW), _full((RAW, H)), _full((1, H)),
              _full((H, HP)), _full((1, HP)), _full((H, HP)), _full((1, HP))],
    out_specs=[_rows(HP), _rows(HP)],
    out_shape=[jax.ShapeDtypeStruct((N, HP), _bf16)] * 2,
)

_mid = pl.pallas_call(
    _mid_body,
    grid=(NB,),
    in_specs=[_rows(HP), _rows(HP),
              _full((HP, H)), _full((HP, H)), _full((1, H)),
              _full((H, HP)), _full((1, HP)), _full((H, HP)), _full((1, HP))],
    out_specs=[_rows(HP), _rows(HP)],
    out_shape=[jax.ShapeDtypeStruct((N, HP), _bf16)] * 2,
)

_readout = pl.pallas_call(
    _readout_body,
    grid=(NB,),
    in_specs=[_rows(HP), _rows(HP),
              _full((HP, H)), _full((HP, H)), _full((1, H)),
              _full((H, 128)), _full((1, 128)), _rows(1)],
    out_specs=pl.BlockSpec((16, 128), lambda i: (0, 0)),
    out_shape=jax.ShapeDtypeStruct((16, 128), _f32),
)


def _split_cols(w, b):
  pad = jnp.zeros((H, HP - HALF), _bf16)
  bpad = jnp.zeros((1, HP - HALF), _f32)
  wb16 = w.astype(_bf16)
  w0 = jnp.concatenate([wb16[:, :HALF], pad], axis=1)
  w1 = jnp.concatenate([wb16[:, HALF:], pad], axis=1)
  b0 = jnp.concatenate([b[:HALF][None], bpad], axis=1)
  b1 = jnp.concatenate([b[HALF:][None], bpad], axis=1)
  return w0, b0, w1, b1


def _split_rows(w):
  pad = jnp.zeros((HP - HALF, H), _bf16)
  wb16 = w.astype(_bf16)
  wa = jnp.concatenate([wb16[:HALF, :], pad], axis=0)
  wb = jnp.concatenate([wb16[HALF:, :], pad], axis=0)
  return wa, wb


def kernel(x, edge_index, graph_ids, W_lift, b_lift, W_ro, b_ro,
           W_msg0, b_msg0, W_out0, b_out0,
           W_msg1, b_msg1, W_out1, b_out1,
           W_msg2, b_msg2, W_out2, b_out2):
  src3 = edge_index[0].reshape(NS, NCH, K)
  dst3 = edge_index[1].reshape(NS, NCH, K)
  zrows = jnp.zeros((RPT, HP), _bf16)

  w00, b00, w01, b01 = _split_cols(W_msg0, b_msg0)
  w10, b10, w11, b11 = _split_cols(W_msg1, b_msg1)
  w20, b20, w21, b21 = _split_cols(W_msg2, b_msg2)
  wa0, wb0 = _split_rows(W_out0)
  wa1, wb1 = _split_rows(W_out1)
  wa2, wb2 = _split_rows(W_out2)

  wro = jnp.zeros((H, 128), _f32).at[:, :C].set(W_ro)
  bro = jnp.zeros((1, 128), _f32).at[0, :C].set(b_ro)

  m0, m1 = _lift_msg(x, W_lift.astype(_bf16), b_lift.reshape(1, H),
                     w00, b00, w01, b01)
  a0, a1 = _scatter_gather(src3, dst3, zrows, m0, m1)

  m0, m1 = _mid(a0, a1, wa0, wb0, b_out0.reshape(1, H), w10, b10, w11, b11)
  a0, a1 = _scatter_gather(src3, dst3, zrows, m0, m1)

  m0, m1 = _mid(a0, a1, wa1, wb1, b_out1.reshape(1, H), w20, b20, w21, b21)
  a0, a1 = _scatter_gather(src3, dst3, zrows, m0, m1)

  acc = _readout(a0, a1, wa2, wb2, b_out2.reshape(1, H), wro, bro,
                 graph_ids.reshape(N, 1))
  return acc[:B, :C]
